# baseline (device time: 10727 ns/iter reference)
import jax
import jax.numpy as jnp
from jax import lax
from jax.experimental import pallas as pl
from jax.experimental.pallas import tpu as pltpu

N_DEV = 4


def kernel(x, router_W, route_idx, expert_W):
    m, d = x.shape
    e_per, _, h = expert_W.shape
    n_exp = N_DEV * e_per
    rows = e_per * d

    def body(x_ref, rw_ref, idx_ref, ew_ref, out_ref,
             w_full, send_sems, recv_sems):
        my_pos = lax.axis_index("i")

        barrier_sem = pltpu.get_barrier_semaphore()
        for delta in range(1, N_DEV):
            pl.semaphore_signal(
                barrier_sem, inc=1,
                device_id=((my_pos + delta) % N_DEV,),
                device_id_type=pl.DeviceIdType.MESH,
            )

        my_rows = pl.ds(my_pos * rows, rows)
        w_full[my_rows, :] = ew_ref[...].astype(jnp.bfloat16).reshape(rows, h)

        pl.semaphore_wait(barrier_sem, N_DEV - 1)

        rdmas = {}
        for delta in [2, 1, N_DEV - 1]:
            rdma = pltpu.make_async_remote_copy(
                src_ref=w_full.at[my_rows],
                dst_ref=w_full.at[my_rows],
                send_sem=send_sems.at[delta - 1],
                recv_sem=recv_sems.at[delta - 1],
                device_id=((my_pos + delta) % N_DEV,),
                device_id_type=pl.DeviceIdType.MESH,
            )
            rdma.start()
            rdmas[delta] = rdma

        xv = x_ref[...]
        scores = jnp.dot(xv, rw_ref[...],
                         preferred_element_type=jnp.float32)
        s_max = jnp.max(scores, axis=1, keepdims=True)
        probs = jnp.exp(scores - s_max)
        probs = probs / jnp.sum(probs, axis=1, keepdims=True)

        idx = idx_ref[...]
        eio = lax.broadcasted_iota(jnp.int32, (m, n_exp), 1)
        oh0 = eio == idx[:, 0:1]
        oh1 = eio == idx[:, 1:2]
        p0 = jnp.sum(jnp.where(oh0, probs, 0.0), axis=1, keepdims=True)
        p1 = jnp.sum(jnp.where(oh1, probs, 0.0), axis=1, keepdims=True)
        gates = (jnp.where(oh0, p0, 0.0) + jnp.where(oh1, p1, 0.0)) / (p0 + p1)

        xg = jnp.concatenate(
            [gates[:, e:e + 1] * xv for e in range(n_exp)], axis=1,
        ).astype(jnp.bfloat16)

        for delta in [1, N_DEV - 1, 2]:
            rdmas[delta].wait_recv()

        out_ref[...] = jnp.dot(xg, w_full[...],
                               preferred_element_type=jnp.float32)

        for rdma in rdmas.values():
            rdma.wait_send()

    return pl.pallas_call(
        body,
        out_shape=jax.ShapeDtypeStruct((m, h), jnp.float32),
        in_specs=[
            pl.BlockSpec(memory_space=pltpu.VMEM),
            pl.BlockSpec(memory_space=pltpu.VMEM),
            pl.BlockSpec(memory_space=pltpu.VMEM),
            pl.BlockSpec(memory_space=pltpu.VMEM),
        ],
        out_specs=pl.BlockSpec(memory_space=pltpu.VMEM),
        scratch_shapes=[
            pltpu.VMEM((n_exp * d, h), jnp.bfloat16),
            pltpu.SemaphoreType.DMA((N_DEV - 1,)),
            pltpu.SemaphoreType.DMA((N_DEV - 1,)),
        ],
        compiler_params=pltpu.CompilerParams(collective_id=0),
    )(x, router_W, route_idx, expert_W)
